# Initial kernel scaffold; baseline (speedup 1.0000x reference)
import jax
import jax.numpy as jnp
from jax.experimental import pallas as pl


def kernel(nodes, op_scheduled, next_op, skip_token):
    raise NotImplementedError("write your pallas kernel here")


# speedup vs baseline: 12.7108x; 12.7108x over previous
import functools

import jax
import jax.numpy as jnp
from jax import lax
from jax.experimental import pallas as pl
from jax.experimental.pallas import tpu as pltpu
from jax.experimental.pallas import tpu_sc as plsc

B, J, O, D = 1024, 50, 20, 128
NC, NS, L = 2, 16, 16
NW = NC * NS
BPW = B // NW
ROWS_PER_B = J * O


def _make_gather_kernel():
    mesh = plsc.VectorSubcoreMesh(core_axis_name="c", subcore_axis_name="s")

    @functools.partial(
        pl.kernel,
        mesh=mesh,
        out_type=jax.ShapeDtypeStruct((B, J + 1, D), jnp.float32),
        scratch_types=[
            pltpu.VMEM((BPW * J + 64,), jnp.int32),
            pltpu.VMEM((64,), jnp.int32),
            pltpu.VMEM((J + 1, D), jnp.float32),
            pltpu.SemaphoreType.DMA,
        ],
    )
    def gather_kernel(nodes_hbm, nop_hbm, skip_hbm, out_hbm,
                      nop_v, idx_v, buf_v, sem):
        wid = lax.axis_index("s") * NC + lax.axis_index("c")
        base_b = wid * BPW

        pltpu.sync_copy(nop_hbm.at[pl.ds(wid * (BPW * J), BPW * J)],
                        nop_v.at[pl.ds(0, BPW * J)])
        pltpu.sync_copy(skip_hbm, buf_v.at[pl.ds(0, 1)])

        lane = lax.broadcasted_iota(jnp.int32, (L,), 0)

        def body(lb, carry):
            b = base_b + lb
            for c in range(4):
                j = c * L + lane
                nop = nop_v[pl.ds(lb * J + c * L, L)]
                idx_v[pl.ds(c * L, L)] = b * ROWS_PER_B + j * O + nop
            pltpu.async_copy(nodes_hbm.at[idx_v.at[pl.ds(0, J)]],
                             buf_v.at[pl.ds(1, J)], sem).wait()
            pltpu.sync_copy(buf_v, out_hbm.at[b])
            return carry

        lax.fori_loop(0, BPW, body, 0)

    return gather_kernel


_gather = _make_gather_kernel()


def kernel(nodes, op_scheduled, next_op, skip_token):
    table = nodes.reshape(B * J * O, D)
    nop = next_op.reshape(B * J).astype(jnp.int32)
    skip = skip_token.reshape(1, D)
    return _gather(table, nop, skip)


# speedup vs baseline: 15.6034x; 1.2276x over previous
import functools

import jax
import jax.numpy as jnp
from jax import lax
from jax.experimental import pallas as pl
from jax.experimental.pallas import tpu as pltpu
from jax.experimental.pallas import tpu_sc as plsc

B, J, O, D = 1024, 50, 20, 128
NC, NS, L = 2, 16, 16
NW = NC * NS
BPW = B // NW
ROWS_PER_B = J * O
NB = 8
G = 4


def _make_gather_kernel():
    mesh = plsc.VectorSubcoreMesh(core_axis_name="c", subcore_axis_name="s")

    scratch = (
        [pltpu.VMEM((BPW * J + 64,), jnp.int32)]
        + [pltpu.VMEM((64,), jnp.int32) for _ in range(NB)]
        + [pltpu.VMEM((J + 1, D), jnp.float32) for _ in range(NB)]
        + [pltpu.SemaphoreType.DMA, pltpu.SemaphoreType.DMA]
    )

    @functools.partial(
        pl.kernel,
        mesh=mesh,
        out_type=jax.ShapeDtypeStruct((B, J + 1, D), jnp.float32),
        scratch_types=scratch,
    )
    def gather_kernel(nodes_hbm, nop_hbm, skip_hbm, out_hbm, nop_v, *rest):
        idx_rings = rest[:NB]
        bufs = rest[NB:2 * NB]
        gsem, wsem = rest[2 * NB], rest[2 * NB + 1]

        wid = lax.axis_index("s") * NC + lax.axis_index("c")
        base_b = wid * BPW

        pltpu.sync_copy(nop_hbm.at[pl.ds(wid * (BPW * J), BPW * J)],
                        nop_v.at[pl.ds(0, BPW * J)])
        for p in range(NB):
            pltpu.sync_copy(skip_hbm, bufs[p].at[pl.ds(0, 1)])

        lane = lax.broadcasted_iota(jnp.int32, (L,), 0)

        ghandles = {}
        whandles = {}
        for i in range(BPW + G):
            if i < BPW:
                p = i % NB
                if i >= NB:
                    whandles[i - NB].wait()
                b = base_b + i
                for c in range(4):
                    nop = nop_v[pl.ds(i * J + c * L, L)]
                    jj = c * L + lane
                    idx_rings[p][pl.ds(c * L, L)] = (
                        b * ROWS_PER_B + jj * O + nop)
                ghandles[i] = pltpu.async_copy(
                    nodes_hbm.at[idx_rings[p].at[pl.ds(0, J)]],
                    bufs[p].at[pl.ds(1, J)], gsem)
            if i >= G:
                k = i - G
                ghandles[k].wait()
                whandles[k] = pltpu.async_copy(
                    bufs[k % NB], out_hbm.at[base_b + k], wsem)
        for k in range(BPW - NB, BPW):
            whandles[k].wait()

    return gather_kernel


_gather = _make_gather_kernel()


def kernel(nodes, op_scheduled, next_op, skip_token):
    table = nodes.reshape(B * J * O, D)
    nop = next_op.reshape(B * J).astype(jnp.int32)
    skip = skip_token.reshape(1, D)
    return _gather(table, nop, skip)


# speedup vs baseline: 26.0133x; 1.6672x over previous
import functools

import jax
import jax.numpy as jnp
from jax import lax
from jax.experimental import pallas as pl
from jax.experimental.pallas import tpu as pltpu
from jax.experimental.pallas import tpu_sc as plsc

B, J, O, D = 1024, 50, 20, 128
NC, NS, L = 2, 16, 16
NW = NC * NS
BPW = B // NW
ROWS_PER_B = J * O
NB = 8
G = 4


def _make_gather_kernel():
    mesh = plsc.VectorSubcoreMesh(core_axis_name="c", subcore_axis_name="s")

    scratch = (
        [pltpu.VMEM((J * BPW,), jnp.int32)]
        + [pltpu.VMEM((BPW,), jnp.int32) for _ in range(NB)]
        + [pltpu.VMEM((BPW, D), jnp.float32) for _ in range(NB)]
        + [pltpu.VMEM((BPW, D), jnp.float32)]
        + [pltpu.SemaphoreType.DMA, pltpu.SemaphoreType.DMA,
           pltpu.SemaphoreType.DMA]
    )

    @functools.partial(
        pl.kernel,
        mesh=mesh,
        out_type=jax.ShapeDtypeStruct((J + 1, B, D), jnp.float32),
        scratch_types=scratch,
    )
    def gather_kernel(nodes_hbm, nop_hbm, skip_hbm, out_hbm, nop_v, *rest):
        idx_rings = rest[:NB]
        bufs = rest[NB:2 * NB]
        skip_v = rest[2 * NB]
        gsem, wsem, nsem = rest[2 * NB + 1:2 * NB + 4]

        wid = lax.axis_index("s") * NC + lax.axis_index("c")
        base_b = wid * BPW

        nh = [
            pltpu.async_copy(nop_hbm.at[pl.ds(j * B + base_b, BPW)],
                             nop_v.at[pl.ds(j * BPW, BPW)], nsem)
            for j in range(J)
        ]
        pltpu.sync_copy(skip_hbm, skip_v)
        pltpu.sync_copy(skip_v, out_hbm.at[0, pl.ds(base_b, BPW)])
        for h in nh:
            h.wait()

        lane = lax.broadcasted_iota(jnp.int32, (L,), 0)

        ghandles = {}
        whandles = {}
        for i in range(J + G):
            if i < J:
                p = i % NB
                if i >= NB:
                    whandles[i - NB].wait()
                for c in range(2):
                    k = c * L + lane
                    nop = nop_v[pl.ds(i * BPW + c * L, L)]
                    idx_rings[p][pl.ds(c * L, L)] = (
                        (base_b + k) * ROWS_PER_B + i * O + nop)
                ghandles[i] = pltpu.async_copy(
                    nodes_hbm.at[idx_rings[p]], bufs[p], gsem)
            if i >= G:
                k = i - G
                ghandles[k].wait()
                whandles[k] = pltpu.async_copy(
                    bufs[k % NB], out_hbm.at[k + 1, pl.ds(base_b, BPW)], wsem)
        for k in range(J - NB, J):
            whandles[k].wait()

    return gather_kernel


_gather = _make_gather_kernel()


def kernel(nodes, op_scheduled, next_op, skip_token):
    table = nodes.reshape(B * J * O, D)
    nop = next_op.astype(jnp.int32).T.reshape(J * B)
    skip = jnp.broadcast_to(skip_token.reshape(1, D), (BPW, D))
    out_t = _gather(table, nop, skip)
    return out_t.transpose(1, 0, 2)


# speedup vs baseline: 26.2977x; 1.0109x over previous
import functools

import jax
import jax.numpy as jnp
from jax import lax
from jax.experimental import pallas as pl
from jax.experimental.pallas import tpu as pltpu
from jax.experimental.pallas import tpu_sc as plsc

B, J, O, D = 1024, 50, 20, 128
NC, NS, L = 2, 16, 16
NW = NC * NS
BPW = B // NW
ROWS_PER_B = J * O
NB = 8
G = 4


def _make_gather_kernel():
    mesh = plsc.VectorSubcoreMesh(core_axis_name="c", subcore_axis_name="s")

    scratch = (
        [pltpu.VMEM((J + 6, 128), jnp.int32)]
        + [pltpu.VMEM((BPW,), jnp.int32) for _ in range(NB)]
        + [pltpu.VMEM((BPW, D), jnp.float32) for _ in range(NB)]
        + [pltpu.VMEM((BPW, D), jnp.float32)]
        + [pltpu.SemaphoreType.DMA, pltpu.SemaphoreType.DMA,
           pltpu.SemaphoreType.DMA]
    )

    @functools.partial(
        pl.kernel,
        mesh=mesh,
        out_type=jax.ShapeDtypeStruct((J + 1, B, D), jnp.float32),
        scratch_types=scratch,
    )
    def gather_kernel(nodes_hbm, nop_hbm, skip_hbm, out_hbm, nop_v, *rest):
        idx_rings = rest[:NB]
        bufs = rest[NB:2 * NB]
        skip_v = rest[2 * NB]
        gsem, wsem, nsem = rest[2 * NB + 1:2 * NB + 4]

        wid = lax.axis_index("s") * NC + lax.axis_index("c")
        base_b = wid * BPW
        col = base_b % 128
        cb = (base_b // 128) * 128

        nh = [
            pltpu.async_copy(
                nop_hbm.at[pl.ds(q * 8, 8 if q < 6 else 2),
                           pl.ds(cb, 128)],
                nop_v.at[pl.ds(q * 8, 8 if q < 6 else 2)], nsem)
            for q in range(7)
        ]
        pltpu.sync_copy(skip_hbm, skip_v.at[pl.ds(0, 1)])
        chunks = [skip_v[0, pl.ds(c * L, L)] for c in range(D // L)]
        for r in range(1, BPW):
            for c in range(D // L):
                skip_v[r, pl.ds(c * L, L)] = chunks[c]
        pltpu.sync_copy(skip_v, out_hbm.at[0, pl.ds(base_b, BPW)])
        for h in nh:
            h.wait()

        lane = lax.broadcasted_iota(jnp.int32, (L,), 0)

        ghandles = {}
        whandles = {}
        for i in range(J + G):
            if i < J:
                p = i % NB
                if i >= NB:
                    whandles[i - NB].wait()
                for c in range(2):
                    k = c * L + lane
                    nop = nop_v[i, pl.ds(col + c * L, L)]
                    idx_rings[p][pl.ds(c * L, L)] = (
                        (base_b + k) * ROWS_PER_B + i * O + nop)
                ghandles[i] = pltpu.async_copy(
                    nodes_hbm.at[idx_rings[p]], bufs[p], gsem)
            if i >= G:
                k = i - G
                ghandles[k].wait()
                whandles[k] = pltpu.async_copy(
                    bufs[k % NB], out_hbm.at[k + 1, pl.ds(base_b, BPW)], wsem)
        for k in range(J - NB, J):
            whandles[k].wait()

    return gather_kernel


_gather = _make_gather_kernel()


def kernel(nodes, op_scheduled, next_op, skip_token):
    table = nodes.reshape(B * J * O, D)
    nop = next_op.astype(jnp.int32).T
    skip = skip_token.reshape(1, D)
    out_t = _gather(table, nop, skip)
    return out_t.transpose(1, 0, 2)


# speedup vs baseline: 26.8622x; 1.0215x over previous
import functools

import jax
import jax.numpy as jnp
from jax import lax
from jax.experimental import pallas as pl
from jax.experimental.pallas import tpu as pltpu
from jax.experimental.pallas import tpu_sc as plsc

B, J, O, D = 1024, 50, 20, 128
NC, NS, L = 2, 16, 16
NW = NC * NS
BPW = B // NW
ROWS_PER_B = J * O
M = 2
NI = J // M
NB = 8
G = 4


def _make_gather_kernel():
    mesh = plsc.VectorSubcoreMesh(core_axis_name="c", subcore_axis_name="s")

    scratch = (
        [pltpu.VMEM((J + 6, 128), jnp.int32)]
        + [pltpu.VMEM((M * BPW,), jnp.int32) for _ in range(NB)]
        + [pltpu.VMEM((M * BPW, D), jnp.float32) for _ in range(NB)]
        + [pltpu.VMEM((BPW, D), jnp.float32)]
        + [pltpu.SemaphoreType.DMA, pltpu.SemaphoreType.DMA,
           pltpu.SemaphoreType.DMA]
    )

    @functools.partial(
        pl.kernel,
        mesh=mesh,
        out_type=jax.ShapeDtypeStruct((J + 1, B, D), jnp.float32),
        scratch_types=scratch,
    )
    def gather_kernel(nodes_hbm, nop_hbm, skip_hbm, out_hbm, nop_v, *rest):
        idx_rings = rest[:NB]
        bufs = rest[NB:2 * NB]
        skip_v = rest[2 * NB]
        gsem, wsem, nsem = rest[2 * NB + 1:2 * NB + 4]

        wid = lax.axis_index("s") * NC + lax.axis_index("c")
        base_b = wid * BPW
        col = base_b % 128
        cb = (base_b // 128) * 128

        nh = [
            pltpu.async_copy(
                nop_hbm.at[pl.ds(q * 8, 8 if q < 6 else 2),
                           pl.ds(cb, 128)],
                nop_v.at[pl.ds(q * 8, 8 if q < 6 else 2)], nsem)
            for q in range(7)
        ]
        pltpu.sync_copy(skip_hbm, skip_v.at[pl.ds(0, 1)])
        chunks = [skip_v[0, pl.ds(c * L, L)] for c in range(D // L)]
        for r in range(1, BPW):
            for c in range(D // L):
                skip_v[r, pl.ds(c * L, L)] = chunks[c]
        pltpu.sync_copy(skip_v, out_hbm.at[0, pl.ds(base_b, BPW)])
        for h in nh:
            h.wait()

        lane = lax.broadcasted_iota(jnp.int32, (L,), 0)

        ghandles = {}
        whandles = {}
        for i in range(NI + G):
            if i < NI:
                p = i % NB
                if i >= NB:
                    for m in range(M):
                        whandles[(i - NB, m)].wait()
                for m in range(M):
                    j = i * M + m
                    for c in range(2):
                        k = c * L + lane
                        nop = nop_v[j, pl.ds(col + c * L, L)]
                        idx_rings[p][pl.ds(m * BPW + c * L, L)] = (
                            (base_b + k) * ROWS_PER_B + j * O + nop)
                ghandles[i] = pltpu.async_copy(
                    nodes_hbm.at[idx_rings[p]], bufs[p], gsem)
            if i >= G:
                k = i - G
                ghandles[k].wait()
                for m in range(M):
                    whandles[(k, m)] = pltpu.async_copy(
                        bufs[k % NB].at[pl.ds(m * BPW, BPW)],
                        out_hbm.at[k * M + m + 1, pl.ds(base_b, BPW)], wsem)
        for k in range(NI - NB, NI):
            for m in range(M):
                whandles[(k, m)].wait()

    return gather_kernel


_gather = _make_gather_kernel()


def kernel(nodes, op_scheduled, next_op, skip_token):
    table = nodes.reshape(B * J * O, D)
    nop = next_op.astype(jnp.int32).T
    skip = skip_token.reshape(1, D)
    out_t = _gather(table, nop, skip)
    return out_t.transpose(1, 0, 2)
